# baseline (speedup 1.0000x reference)
import functools

import jax
import jax.numpy as jnp
from jax import lax
from jax.experimental import pallas as pl
from jax.experimental.pallas import tpu as pltpu
from jax.experimental.pallas import tpu_sc as plsc

N_NODES = 10000
N_INC = 160000
VOCAB = 128
HIDDEN = 256
N_CYCLES = 10000

NC = 2
NS = 16
L = 16
NW = NC * NS
INC_W = N_INC // NW
CHUNK = 128
NCHUNK = (INC_W + CHUNK - 1) // CHUNK
SLAB = NCHUNK * CHUNK
NGRP = SLAB // L
CNT_REAL = N_CYCLES * VOCAB
SCRAP = CNT_REAL
ZCH = 8192
ZPT = 10
CNT_PAD = NS * ZPT * ZCH
OUT_W = CNT_REAL // NS

K = 40
NCK = INC_W // K

_sc_mesh = plsc.VectorSubcoreMesh(core_axis_name="c", subcore_axis_name="s")
_sc_params = pltpu.CompilerParams(needs_layout_passes=False)


@functools.partial(
    pl.kernel,
    out_type=(
        jax.ShapeDtypeStruct((N_INC,), jnp.int32),
        jax.ShapeDtypeStruct((NC * CNT_REAL,), jnp.float32)
    ),
    mesh=_sc_mesh,
    scratch_types=[
        pltpu.VMEM((N_NODES,), jnp.int32),
        pltpu.VMEM((SLAB,), jnp.int32),
        pltpu.VMEM((SLAB,), jnp.int32),
        pltpu.VMEM((SLAB,), jnp.int32),
        pltpu.VMEM((NCHUNK, CHUNK), jnp.int32),
        pltpu.VMEM((CHUNK,), jnp.float32),
        pltpu.VMEM((ZCH,), jnp.float32),
        pltpu.VMEM_SHARED((CNT_PAD,), jnp.float32),
    ],
    compiler_params=_sc_params,
)
def _sc_index_counts(x_hbm, ni_hbm, cy_hbm, idx_hbm, cnt_hbm,
                     x_v, ni_v, cy_v, v_v, flat_v, ones_v, zero_v, cnt_sh):
    c = lax.axis_index("c")
    s = lax.axis_index("s")
    w = c * NS + s
    base = w * INC_W

    pltpu.sync_copy(x_hbm, x_v)
    pltpu.sync_copy(ni_hbm.at[pl.ds(base, INC_W)], ni_v.at[pl.ds(0, INC_W)])
    pltpu.sync_copy(cy_hbm.at[pl.ds(base, INC_W)], cy_v.at[pl.ds(0, INC_W)])

    def _fill_ones(i, _):
        ones_v[pl.ds(i * L, L)] = jnp.full((L,), 1.0, jnp.float32)
        return 0
    lax.fori_loop(0, CHUNK // L, _fill_ones, 0)

    def _fill_zero(i, _):
        zero_v[pl.ds(i * L, L)] = jnp.zeros((L,), jnp.float32)
        return 0
    lax.fori_loop(0, ZCH // L, _fill_zero, 0)

    def _zero_cnt(j, _):
        pltpu.sync_copy(zero_v, cnt_sh.at[pl.ds((s * ZPT + j) * ZCH, ZCH)])
        return 0
    lax.fori_loop(0, ZPT, _zero_cnt, 0)
    plsc.subcore_barrier()

    lane = lax.iota(jnp.int32, L)

    def _grp(g, _):
        off = g * L
        ni = ni_v[pl.ds(off, L)]
        ni = jnp.clip(ni, 0, N_NODES - 1)
        cy = cy_v[pl.ds(off, L)]
        v = plsc.load_gather(x_v, [ni])
        flat = cy * VOCAB + v
        flat = jnp.where(off + lane < INC_W, flat, SCRAP)
        v_v[pl.ds(off, L)] = v
        row = g // (CHUNK // L)
        col = (g % (CHUNK // L)) * L
        flat_v[row, pl.ds(col, L)] = flat
        return 0
    lax.fori_loop(0, NGRP, _grp, 0)

    def _scat(j, _):
        pltpu.sync_copy(ones_v, cnt_sh.at[flat_v.at[j]], add=True)
        return 0
    lax.fori_loop(0, NCHUNK, _scat, 0)
    plsc.subcore_barrier()

    pltpu.sync_copy(cnt_sh.at[pl.ds(s * OUT_W, OUT_W)],
                    cnt_hbm.at[pl.ds(c * CNT_REAL + s * OUT_W, OUT_W)])
    pltpu.sync_copy(v_v.at[pl.ds(0, INC_W)], idx_hbm.at[pl.ds(base, INC_W)])


_RB = 1000


def _tc_seg_body(cnt_ref, emb_ref, seg_ref):
    csum = cnt_ref[0] + cnt_ref[1]
    seg_ref[...] = jnp.dot(csum, emb_ref[...],
                           preferred_element_type=jnp.float32)


def _tc_seg(counts3, emb_table):
    return pl.pallas_call(
        _tc_seg_body,
        grid=(N_CYCLES // _RB,),
        in_specs=[
            pl.BlockSpec((NC, _RB, VOCAB), lambda i: (0, i, 0)),
            pl.BlockSpec((VOCAB, HIDDEN), lambda i: (0, 0)),
        ],
        out_specs=pl.BlockSpec((_RB, HIDDEN), lambda i: (i, 0)),
        out_shape=jax.ShapeDtypeStruct((N_CYCLES, HIDDEN), jnp.float32),
    )(counts3, emb_table)


@functools.partial(
    pl.kernel,
    out_type=jax.ShapeDtypeStruct((N_INC, HIDDEN), jnp.float32),
    mesh=_sc_mesh,
    scratch_types=[
        pltpu.VMEM((INC_W,), jnp.int32),
        pltpu.VMEM((INC_W,), jnp.int32),
        pltpu.VMEM((VOCAB * HIDDEN,), jnp.float32),
        pltpu.VMEM((4, K, HIDDEN), jnp.float32),
        pltpu.SemaphoreType.DMA,
        pltpu.SemaphoreType.DMA,
        pltpu.SemaphoreType.DMA,
        pltpu.SemaphoreType.DMA,
        pltpu.SemaphoreType.DMA,
        pltpu.SemaphoreType.DMA,
        pltpu.SemaphoreType.DMA,
        pltpu.SemaphoreType.DMA,
    ],
    compiler_params=_sc_params,
)
def _sc_combine(idx_hbm, cy_hbm, emb2_hbm, seg_hbm, out_hbm,
                idx_v, cy_v, emb_v, b_v,
                sb0, sb1, sb2, sb3, so0, so1, so2, so3):
    c = lax.axis_index("c")
    s = lax.axis_index("s")
    base = (c * NS + s) * INC_W
    sems_b = (sb0, sb1, sb2, sb3)
    sems_o = (so0, so1, so2, so3)

    pltpu.sync_copy(idx_hbm.at[pl.ds(base, INC_W)], idx_v)
    pltpu.sync_copy(cy_hbm.at[pl.ds(base, INC_W)], cy_v)
    pltpu.sync_copy(emb2_hbm, emb_v)

    def _fire_b(j, slot):
        pltpu.async_copy(seg_hbm.at[cy_v.at[pl.ds(j * K, K)]],
                         b_v.at[slot], sems_b[slot])

    def _wait_b(j, slot):
        pltpu.make_async_copy(seg_hbm.at[cy_v.at[pl.ds(j * K, K)]],
                              b_v.at[slot], sems_b[slot]).wait()

    def _fire_s(j, slot):
        pltpu.async_copy(b_v.at[slot], out_hbm.at[pl.ds(base + j * K, K)],
                         sems_o[slot])

    def _wait_s(j, slot):
        pltpu.make_async_copy(b_v.at[slot],
                              out_hbm.at[pl.ds(base + j * K, K)],
                              sems_o[slot]).wait()

    lane = lax.iota(jnp.int32, L)
    UC = 8

    def _combine(q, slot):
        for goff, masked in ((0, False), (L, False), (K - L, True)):
            vv = idx_v[pl.ds(q * K + goff, L)]
            vvb = vv * HIDDEN
            rowv = lane + goff
            msk = lane >= (3 * L - K) if masked else None
            slotv = jnp.full((L,), slot, jnp.int32)

            def _colblk(t, _):
                t0 = t * UC
                evb = vvb + t0
                cvb = jnp.full((L,), 0, jnp.int32) + t0
                for u in range(UC):
                    cv = cvb + u
                    e = plsc.load_gather(emb_v, [evb + u])
                    b = plsc.load_gather(b_v, [slotv, rowv, cv])
                    plsc.store_scatter(b_v, [slotv, rowv, cv], e + b,
                                       mask=msk)
                return 0
            lax.fori_loop(0, HIDDEN // UC, _colblk, 0)

    _fire_b(0, 0)
    _fire_b(1, 1)

    def _step(q, slot):
        @pl.when(q < NCK)
        def _():
            @pl.when(q >= 2)
            def _():
                _wait_s(q - 2, (slot + 2) % 4)

            @pl.when(q + 2 < NCK)
            def _():
                _fire_b(q + 2, (slot + 2) % 4)

            _wait_b(q, slot)
            _combine(q, slot)
            _fire_s(q, slot)

    def _quad(t, _):
        for u in range(4):
            _step(4 * t + u, u)
        return 0
    lax.fori_loop(0, (NCK + 3) // 4, _quad, 0)

    _wait_s(NCK - 2, (NCK - 2) % 4)
    _wait_s(NCK - 1, (NCK - 1) % 4)


@jax.jit
def kernel(x, node_index, cycle_id, emb_table, epsilon):
    x = x.astype(jnp.int32)
    node_index = node_index.astype(jnp.int32)
    cycle_id = cycle_id.astype(jnp.int32)
    idx, counts = _sc_index_counts(x, node_index, cycle_id)
    seg = _tc_seg(counts.reshape(NC, N_CYCLES, VOCAB), emb_table)
    emb2 = ((1.0 + epsilon) * emb_table).reshape(VOCAB * HIDDEN)
    return _sc_combine(idx, cycle_id, emb2, seg)


# speedup vs baseline: 5.0102x; 5.0102x over previous
import functools

import jax
import jax.numpy as jnp
from jax import lax
from jax.experimental import pallas as pl
from jax.experimental.pallas import tpu as pltpu
from jax.experimental.pallas import tpu_sc as plsc

N_NODES = 10000
N_INC = 160000
VOCAB = 128
HIDDEN = 256
N_CYCLES = 10000

NC = 2
NS = 16
L = 16
NW = NC * NS
INC_W = N_INC // NW
CHUNK = 128
NCHUNK = (INC_W + CHUNK - 1) // CHUNK
SLAB = NCHUNK * CHUNK
NGRP = SLAB // L
CNT_REAL = N_CYCLES * VOCAB
SCRAP = CNT_REAL
ZCH = 8192
ZPT = 10
CNT_PAD = NS * ZPT * ZCH
OUT_W = CNT_REAL // NS

K = 40
NCK = INC_W // K

_sc_mesh = plsc.VectorSubcoreMesh(core_axis_name="c", subcore_axis_name="s")
_sc_params = pltpu.CompilerParams(needs_layout_passes=False)


@functools.partial(
    pl.kernel,
    out_type=(
        jax.ShapeDtypeStruct((N_INC,), jnp.int32),
        jax.ShapeDtypeStruct((NC * CNT_REAL,), jnp.float32)
    ),
    mesh=_sc_mesh,
    scratch_types=[
        pltpu.VMEM((N_NODES,), jnp.int32),
        pltpu.VMEM((SLAB,), jnp.int32),
        pltpu.VMEM((SLAB,), jnp.int32),
        pltpu.VMEM((SLAB,), jnp.int32),
        pltpu.VMEM((NCHUNK, CHUNK), jnp.int32),
        pltpu.VMEM((CHUNK,), jnp.float32),
        pltpu.VMEM((ZCH,), jnp.float32),
        pltpu.VMEM_SHARED((CNT_PAD,), jnp.float32),
    ],
    compiler_params=_sc_params,
)
def _sc_index_counts(x_hbm, ni_hbm, cy_hbm, idx_hbm, cnt_hbm,
                     x_v, ni_v, cy_v, v_v, flat_v, ones_v, zero_v, cnt_sh):
    c = lax.axis_index("c")
    s = lax.axis_index("s")
    w = c * NS + s
    base = w * INC_W

    pltpu.sync_copy(x_hbm, x_v)
    pltpu.sync_copy(ni_hbm.at[pl.ds(base, INC_W)], ni_v.at[pl.ds(0, INC_W)])
    pltpu.sync_copy(cy_hbm.at[pl.ds(base, INC_W)], cy_v.at[pl.ds(0, INC_W)])

    def _fill_ones(i, _):
        ones_v[pl.ds(i * L, L)] = jnp.full((L,), 1.0, jnp.float32)
        return 0
    lax.fori_loop(0, CHUNK // L, _fill_ones, 0)

    def _fill_zero(i, _):
        zero_v[pl.ds(i * L, L)] = jnp.zeros((L,), jnp.float32)
        return 0
    lax.fori_loop(0, ZCH // L, _fill_zero, 0)

    def _zero_cnt(j, _):
        pltpu.sync_copy(zero_v, cnt_sh.at[pl.ds((s * ZPT + j) * ZCH, ZCH)])
        return 0
    lax.fori_loop(0, ZPT, _zero_cnt, 0)
    plsc.subcore_barrier()

    lane = lax.iota(jnp.int32, L)

    def _grp(g, _):
        off = g * L
        ni = ni_v[pl.ds(off, L)]
        ni = jnp.clip(ni, 0, N_NODES - 1)
        cy = cy_v[pl.ds(off, L)]
        v = plsc.load_gather(x_v, [ni])
        flat = cy * VOCAB + v
        flat = jnp.where(off + lane < INC_W, flat, SCRAP)
        v_v[pl.ds(off, L)] = v
        row = g // (CHUNK // L)
        col = (g % (CHUNK // L)) * L
        flat_v[row, pl.ds(col, L)] = flat
        return 0
    lax.fori_loop(0, NGRP, _grp, 0)

    def _scat(j, _):
        pltpu.sync_copy(ones_v, cnt_sh.at[flat_v.at[j]], add=True)
        return 0
    lax.fori_loop(0, NCHUNK, _scat, 0)
    plsc.subcore_barrier()

    pltpu.sync_copy(cnt_sh.at[pl.ds(s * OUT_W, OUT_W)],
                    cnt_hbm.at[pl.ds(c * CNT_REAL + s * OUT_W, OUT_W)])
    pltpu.sync_copy(v_v.at[pl.ds(0, INC_W)], idx_hbm.at[pl.ds(base, INC_W)])


_RB = 1000


def _tc_seg_body(cnt_ref, emb_ref, seg_ref):
    csum = cnt_ref[0] + cnt_ref[1]
    seg_ref[...] = jnp.dot(csum, emb_ref[...],
                           preferred_element_type=jnp.float32)


def _tc_seg(counts3, emb_table):
    return pl.pallas_call(
        _tc_seg_body,
        grid=(N_CYCLES // _RB,),
        in_specs=[
            pl.BlockSpec((NC, _RB, VOCAB), lambda i: (0, i, 0)),
            pl.BlockSpec((VOCAB, HIDDEN), lambda i: (0, 0)),
        ],
        out_specs=pl.BlockSpec((_RB, HIDDEN), lambda i: (i, 0)),
        out_shape=jax.ShapeDtypeStruct((N_CYCLES, HIDDEN), jnp.float32),
    )(counts3, emb_table)


@functools.partial(
    pl.kernel,
    out_type=jax.ShapeDtypeStruct((N_INC, HIDDEN), jnp.float32),
    mesh=_sc_mesh,
    scratch_types=[
        pltpu.VMEM((INC_W,), jnp.int32),
        pltpu.VMEM((INC_W,), jnp.int32),
        pltpu.VMEM((VOCAB * HIDDEN,), jnp.float32),
        pltpu.VMEM((4, K, HIDDEN), jnp.float32),
        pltpu.SemaphoreType.DMA,
        pltpu.SemaphoreType.DMA,
        pltpu.SemaphoreType.DMA,
        pltpu.SemaphoreType.DMA,
        pltpu.SemaphoreType.DMA,
        pltpu.SemaphoreType.DMA,
        pltpu.SemaphoreType.DMA,
        pltpu.SemaphoreType.DMA,
    ],
    compiler_params=_sc_params,
)
def _sc_combine(idx_hbm, cy_hbm, emb2_hbm, seg_hbm, out_hbm,
                idx_v, cy_v, emb_v, b_v,
                sb0, sb1, sb2, sb3, so0, so1, so2, so3):
    c = lax.axis_index("c")
    s = lax.axis_index("s")
    base = (c * NS + s) * INC_W
    sems_b = (sb0, sb1, sb2, sb3)
    sems_o = (so0, so1, so2, so3)

    pltpu.sync_copy(idx_hbm.at[pl.ds(base, INC_W)], idx_v)
    pltpu.sync_copy(cy_hbm.at[pl.ds(base, INC_W)], cy_v)
    pltpu.sync_copy(emb2_hbm, emb_v)

    def _fire_b(j, slot):
        pltpu.async_copy(seg_hbm.at[cy_v.at[pl.ds(j * K, K)]],
                         b_v.at[slot], sems_b[slot])

    def _wait_b(j, slot):
        pltpu.make_async_copy(seg_hbm.at[cy_v.at[pl.ds(j * K, K)]],
                              b_v.at[slot], sems_b[slot]).wait()

    def _fire_s(j, slot):
        pltpu.async_copy(b_v.at[slot], out_hbm.at[pl.ds(base + j * K, K)],
                         sems_o[slot])

    def _wait_s(j, slot):
        pltpu.make_async_copy(b_v.at[slot],
                              out_hbm.at[pl.ds(base + j * K, K)],
                              sems_o[slot]).wait()

    lane = lax.iota(jnp.int32, L)

    def _combine(q, slot):
        def _row(r, _):
            bcast = plsc.load_gather(
                idx_v, [jnp.full((L,), q * K + r, jnp.int32)]) * HIDDEN
            for cg in range(HIDDEN // L):
                e = plsc.load_gather(emb_v, [bcast + (cg * L) + lane])
                b_v[slot, r, pl.ds(cg * L, L)] = (
                    b_v[slot, r, pl.ds(cg * L, L)] + e)
            return 0
        lax.fori_loop(0, K, _row, 0)

    _fire_b(0, 0)
    _fire_b(1, 1)

    def _step(q, slot):
        @pl.when(q < NCK)
        def _():
            @pl.when(q >= 2)
            def _():
                _wait_s(q - 2, (slot + 2) % 4)

            @pl.when(q + 2 < NCK)
            def _():
                _fire_b(q + 2, (slot + 2) % 4)

            _wait_b(q, slot)
            _combine(q, slot)
            _fire_s(q, slot)

    def _quad(t, _):
        for u in range(4):
            _step(4 * t + u, u)
        return 0
    lax.fori_loop(0, (NCK + 3) // 4, _quad, 0)

    _wait_s(NCK - 2, (NCK - 2) % 4)
    _wait_s(NCK - 1, (NCK - 1) % 4)


@jax.jit
def kernel(x, node_index, cycle_id, emb_table, epsilon):
    x = x.astype(jnp.int32)
    node_index = node_index.astype(jnp.int32)
    cycle_id = cycle_id.astype(jnp.int32)
    idx, counts = _sc_index_counts(x, node_index, cycle_id)
    seg = _tc_seg(counts.reshape(NC, N_CYCLES, VOCAB), emb_table)
    emb2 = ((1.0 + epsilon) * emb_table).reshape(VOCAB * HIDDEN)
    return _sc_combine(idx, cycle_id, emb2, seg)


# speedup vs baseline: 8.8566x; 1.7677x over previous
import functools

import jax
import jax.numpy as jnp
from jax import lax
from jax.experimental import pallas as pl
from jax.experimental.pallas import tpu as pltpu
from jax.experimental.pallas import tpu_sc as plsc

N_NODES = 10000
N_INC = 160000
VOCAB = 128
HIDDEN = 256
N_CYCLES = 10000

NC = 2
NS = 16
L = 16
NW = NC * NS
INC_W = N_INC // NW
CHUNK = 128
NCHUNK = (INC_W + CHUNK - 1) // CHUNK
SLAB = NCHUNK * CHUNK
NGRP = SLAB // L
CNT_REAL = N_CYCLES * VOCAB
SCRAP = CNT_REAL
ZCH = 8192
ZPT = 10
CNT_PAD = NS * ZPT * ZCH
OUT_W = CNT_REAL // NS

K = 40
NCK = INC_W // K

_sc_mesh = plsc.VectorSubcoreMesh(core_axis_name="c", subcore_axis_name="s")
_sc_params = pltpu.CompilerParams(needs_layout_passes=False)


@functools.partial(
    pl.kernel,
    out_type=(
        jax.ShapeDtypeStruct((N_INC,), jnp.int32),
        jax.ShapeDtypeStruct((NC * CNT_REAL,), jnp.float32)
    ),
    mesh=_sc_mesh,
    scratch_types=[
        pltpu.VMEM((N_NODES,), jnp.int32),
        pltpu.VMEM((SLAB,), jnp.int32),
        pltpu.VMEM((SLAB,), jnp.int32),
        pltpu.VMEM((SLAB,), jnp.int32),
        pltpu.VMEM((NCHUNK, CHUNK), jnp.int32),
        pltpu.VMEM((CHUNK,), jnp.float32),
        pltpu.VMEM((ZCH,), jnp.float32),
        pltpu.VMEM_SHARED((CNT_PAD,), jnp.float32),
    ],
    compiler_params=_sc_params,
)
def _sc_index_counts(x_hbm, ni_hbm, cy_hbm, idx_hbm, cnt_hbm,
                     x_v, ni_v, cy_v, v_v, flat_v, ones_v, zero_v, cnt_sh):
    c = lax.axis_index("c")
    s = lax.axis_index("s")
    w = c * NS + s
    base = w * INC_W

    pltpu.sync_copy(x_hbm, x_v)
    pltpu.sync_copy(ni_hbm.at[pl.ds(base, INC_W)], ni_v.at[pl.ds(0, INC_W)])
    pltpu.sync_copy(cy_hbm.at[pl.ds(base, INC_W)], cy_v.at[pl.ds(0, INC_W)])

    def _fill_ones(i, _):
        ones_v[pl.ds(i * L, L)] = jnp.full((L,), 1.0, jnp.float32)
        return 0
    lax.fori_loop(0, CHUNK // L, _fill_ones, 0)

    def _fill_zero(i, _):
        zero_v[pl.ds(i * L, L)] = jnp.zeros((L,), jnp.float32)
        return 0
    lax.fori_loop(0, ZCH // L, _fill_zero, 0)

    def _zero_cnt(j, _):
        pltpu.sync_copy(zero_v, cnt_sh.at[pl.ds((s * ZPT + j) * ZCH, ZCH)])
        return 0
    lax.fori_loop(0, ZPT, _zero_cnt, 0)
    plsc.subcore_barrier()

    lane = lax.iota(jnp.int32, L)

    def _grp(g, _):
        off = g * L
        ni = ni_v[pl.ds(off, L)]
        ni = jnp.clip(ni, 0, N_NODES - 1)
        cy = cy_v[pl.ds(off, L)]
        v = plsc.load_gather(x_v, [ni])
        flat = cy * VOCAB + v
        flat = jnp.where(off + lane < INC_W, flat, SCRAP)
        v_v[pl.ds(off, L)] = v
        row = g // (CHUNK // L)
        col = (g % (CHUNK // L)) * L
        flat_v[row, pl.ds(col, L)] = flat
        return 0
    lax.fori_loop(0, NGRP, _grp, 0)

    def _scat(j, _):
        pltpu.sync_copy(ones_v, cnt_sh.at[flat_v.at[j]], add=True)
        return 0
    lax.fori_loop(0, NCHUNK, _scat, 0)
    plsc.subcore_barrier()

    pltpu.sync_copy(cnt_sh.at[pl.ds(s * OUT_W, OUT_W)],
                    cnt_hbm.at[pl.ds(c * CNT_REAL + s * OUT_W, OUT_W)])
    pltpu.sync_copy(v_v.at[pl.ds(0, INC_W)], idx_hbm.at[pl.ds(base, INC_W)])


_RB = 1000


def _tc_seg_body(cnt_ref, emb_ref, seg_ref):
    csum = cnt_ref[0] + cnt_ref[1]
    seg_ref[...] = jnp.dot(csum, emb_ref[...],
                           preferred_element_type=jnp.float32)


def _tc_seg(counts3, emb_table):
    return pl.pallas_call(
        _tc_seg_body,
        grid=(N_CYCLES // _RB,),
        in_specs=[
            pl.BlockSpec((NC, _RB, VOCAB), lambda i: (0, i, 0)),
            pl.BlockSpec((VOCAB, HIDDEN), lambda i: (0, 0)),
        ],
        out_specs=pl.BlockSpec((_RB, HIDDEN), lambda i: (i, 0)),
        out_shape=jax.ShapeDtypeStruct((N_CYCLES, HIDDEN), jnp.float32),
    )(counts3, emb_table)


@functools.partial(
    pl.kernel,
    out_type=jax.ShapeDtypeStruct((N_INC, HIDDEN), jnp.float32),
    mesh=_sc_mesh,
    scratch_types=[
        pltpu.VMEM((INC_W,), jnp.int32),
        pltpu.VMEM((INC_W,), jnp.int32),
        pltpu.VMEM((VOCAB * HIDDEN,), jnp.float32),
        pltpu.VMEM((4, K, HIDDEN), jnp.float32),
        pltpu.SemaphoreType.DMA,
        pltpu.SemaphoreType.DMA,
        pltpu.SemaphoreType.DMA,
        pltpu.SemaphoreType.DMA,
        pltpu.SemaphoreType.DMA,
        pltpu.SemaphoreType.DMA,
        pltpu.SemaphoreType.DMA,
        pltpu.SemaphoreType.DMA,
    ],
    compiler_params=_sc_params,
)
def _sc_combine(idx_hbm, cy_hbm, emb2_hbm, seg_hbm, out_hbm,
                idx_v, cy_v, emb_v, b_v,
                sb0, sb1, sb2, sb3, so0, so1, so2, so3):
    c = lax.axis_index("c")
    s = lax.axis_index("s")
    base = (c * NS + s) * INC_W
    sems_b = (sb0, sb1, sb2, sb3)
    sems_o = (so0, so1, so2, so3)

    pltpu.sync_copy(idx_hbm.at[pl.ds(base, INC_W)], idx_v)
    pltpu.sync_copy(cy_hbm.at[pl.ds(base, INC_W)], cy_v)
    pltpu.sync_copy(emb2_hbm, emb_v)

    def _fire_b(j, slot):
        pltpu.async_copy(seg_hbm.at[cy_v.at[pl.ds(j * K, K)]],
                         b_v.at[slot], sems_b[slot])

    def _wait_b(j, slot):
        pltpu.make_async_copy(seg_hbm.at[cy_v.at[pl.ds(j * K, K)]],
                              b_v.at[slot], sems_b[slot]).wait()

    def _fire_s(j, slot):
        pltpu.async_copy(b_v.at[slot], out_hbm.at[pl.ds(base + j * K, K)],
                         sems_o[slot])

    def _wait_s(j, slot):
        pltpu.make_async_copy(b_v.at[slot],
                              out_hbm.at[pl.ds(base + j * K, K)],
                              sems_o[slot]).wait()

    lane = lax.iota(jnp.int32, L)

    def _combine(q, slot):
        @functools.partial(plsc.parallel_loop, 0, K, unroll=2)
        def _row(r):
            bcast = plsc.load_gather(
                idx_v, [jnp.full((L,), q * K + r, jnp.int32)]) * HIDDEN
            for cg in range(HIDDEN // L):
                e = plsc.load_gather(emb_v, [bcast + (cg * L) + lane])
                plsc.addupdate(b_v.at[slot, r, pl.ds(cg * L, L)], e)

    _fire_b(0, 0)
    _fire_b(1, 1)

    def _step(q, slot):
        @pl.when(q < NCK)
        def _():
            @pl.when(q >= 2)
            def _():
                _wait_s(q - 2, (slot + 2) % 4)

            @pl.when(q + 2 < NCK)
            def _():
                _fire_b(q + 2, (slot + 2) % 4)

            _wait_b(q, slot)
            _combine(q, slot)
            _fire_s(q, slot)

    def _quad(t, _):
        for u in range(4):
            _step(4 * t + u, u)
        return 0
    lax.fori_loop(0, (NCK + 3) // 4, _quad, 0)

    _wait_s(NCK - 2, (NCK - 2) % 4)
    _wait_s(NCK - 1, (NCK - 1) % 4)


@jax.jit
def kernel(x, node_index, cycle_id, emb_table, epsilon):
    x = x.astype(jnp.int32)
    node_index = node_index.astype(jnp.int32)
    cycle_id = cycle_id.astype(jnp.int32)
    idx, counts = _sc_index_counts(x, node_index, cycle_id)
    seg = _tc_seg(counts.reshape(NC, N_CYCLES, VOCAB), emb_table)
    emb2 = ((1.0 + epsilon) * emb_table).reshape(VOCAB * HIDDEN)
    return _sc_combine(idx, cycle_id, emb2, seg)
